# baseline (speedup 1.0000x reference)
import jax
import jax.numpy as jnp
from jax.experimental import pallas as pl

_EPS = 1e-12
_ROWS = 1_000_000
_DIM = 64
_BLOCK_ROWS = 8_000


def _l2norm_body(x_ref, o_ref):
    x = x_ref[...]
    n = jnp.sum(x * x, axis=1, keepdims=True)
    scale = 1.0 / jnp.maximum(jnp.sqrt(n), _EPS)
    o_ref[...] = x * scale


def kernel(weight):
    grid = (_ROWS // _BLOCK_ROWS,)
    return pl.pallas_call(
        _l2norm_body,
        grid=grid,
        in_specs=[pl.BlockSpec((_BLOCK_ROWS, _DIM), lambda i: (i, 0))],
        out_specs=pl.BlockSpec((_BLOCK_ROWS, _DIM), lambda i: (i, 0)),
        out_shape=jax.ShapeDtypeStruct((_ROWS, _DIM), jnp.float32),
    )(weight)


# speedup vs baseline: 1.0022x; 1.0022x over previous
import jax
import jax.numpy as jnp
from jax.experimental import pallas as pl

_EPS = 1e-12
_ROWS = 1_000_000
_DIM = 64
_BLOCK_ROWS = 8_000


def _l2norm_body(x_ref, o_ref):
    x = x_ref[...]
    ones = jnp.ones((_DIM, _DIM), dtype=jnp.float32)
    n = jax.lax.dot(x * x, ones, preferred_element_type=jnp.float32)
    scale = 1.0 / jnp.maximum(jnp.sqrt(n), _EPS)
    o_ref[...] = x * scale


def kernel(weight):
    grid = (_ROWS // _BLOCK_ROWS,)
    return pl.pallas_call(
        _l2norm_body,
        grid=grid,
        in_specs=[pl.BlockSpec((_BLOCK_ROWS, _DIM), lambda i: (i, 0))],
        out_specs=pl.BlockSpec((_BLOCK_ROWS, _DIM), lambda i: (i, 0)),
        out_shape=jax.ShapeDtypeStruct((_ROWS, _DIM), jnp.float32),
    )(weight)
